# Initial kernel scaffold; baseline (speedup 1.0000x reference)
import jax
import jax.numpy as jnp
from jax.experimental import pallas as pl


def kernel(x, edge_index, W_l1, b_l1, W_r1, W_l2, b_l2, W_r2, W3, b3):
    raise NotImplementedError("write your pallas kernel here")


# speedup vs baseline: 3.2138x; 3.2138x over previous
import jax
import jax.numpy as jnp
from jax import lax
from jax.experimental import pallas as pl
from jax.experimental.pallas import tpu as pltpu
from jax.experimental.pallas import tpu_sc as plsc

NC = 2
NS = 16
K = 128

F32 = jnp.float32


def _round_up(a, b):
    return (a + b - 1) // b * b


def _make_sc_layer(npad, epad, feat, compute_deg):
    sp = npad // NS
    q = epad // (NC * NS)
    ch = q // K
    ncc = feat // 16
    npieces = sp // K

    out_type = [jax.ShapeDtypeStruct((NC * npad, feat), F32)]
    if compute_deg:
        out_type.append(jax.ShapeDtypeStruct((NC * npad, feat), F32))

    scratch = [
        pltpu.VMEM((K,), jnp.int32),
        pltpu.VMEM((K,), jnp.int32),
        pltpu.VMEM((K,), jnp.int32),
        pltpu.VMEM((K,), jnp.int32),
        pltpu.VMEM((K, feat), F32),
        pltpu.VMEM((K, feat), F32),
        pltpu.VMEM((K,), jnp.int32),
        pltpu.VMEM_SHARED((npad, feat), F32),
        pltpu.SemaphoreType.DMA,
        pltpu.SemaphoreType.DMA,
        pltpu.SemaphoreType.DMA,
        pltpu.SemaphoreType.DMA,
        pltpu.SemaphoreType.DMA,
        pltpu.SemaphoreType.DMA,
    ]

    def body(y_hbm, src_hbm, dst_hbm, iota_hbm, *rest):
        if compute_deg:
            part_hbm, deg_out = rest[0], rest[1]
            scr = rest[2:]
        else:
            part_hbm = rest[0]
            scr = rest[1:]
        (src0, src1, dst0, dst1, rows0, rows1, stg_idx, acc_sh,
         isem0, isem1, gsem0, gsem1, ssem0, ssem1) = scr
        srcb = (src0, src1)
        dstb = (dst0, dst1)
        rowsb = (rows0, rows1)
        isem = (isem0, isem1)
        gsem = (gsem0, gsem1)
        ssem = (ssem0, ssem1)

        cid = lax.axis_index("c")
        sid = lax.axis_index("s")
        r0 = sid * sp
        abase = cid * (epad // NC) + sid * q

        def set_stg(pr0):
            pltpu.sync_copy(iota_hbm.at[pl.ds(pr0, K)], stg_idx)

        def zero_own_stripe():
            @pl.loop(0, npieces)
            def _(p):
                set_stg(r0 + p * K)
                pltpu.sync_copy(rows1, acc_sh.at[stg_idx])

        def readback(out_arr):
            @pl.loop(0, npieces)
            def _(p):
                pr0 = r0 + p * K
                set_stg(pr0)
                pltpu.sync_copy(acc_sh.at[stg_idx], rows0)
                pltpu.sync_copy(rows0,
                                out_arr.at[pl.ds(cid * npad + pr0, K)])

        zf = jnp.zeros((16,), F32)
        one = jnp.ones((16,), F32)

        @pl.loop(0, K)
        def _(r):
            for cc in range(ncc):
                rows1[r, pl.ds(cc * 16, 16)] = zf
            if compute_deg:
                for cc in range(ncc):
                    rows0[r, pl.ds(cc * 16, 16)] = one

        with jax.named_scope("ph_zero"):
            zero_own_stripe()
        plsc.subcore_barrier()

        if compute_deg:
            with jax.named_scope("ph_deg"):
                @pl.loop(0, ch)
                def _(j):
                    e0 = abase + j * K
                    pltpu.sync_copy(dst_hbm.at[pl.ds(e0, K)], dst0)
                    pltpu.sync_copy(rows0, acc_sh.at[dst0], add=True)

            plsc.subcore_barrier()
            with jax.named_scope("ph_degrb"):
                readback(deg_out)
            plsc.subcore_barrier()
            with jax.named_scope("ph_rezero"):
                zero_own_stripe()
            plsc.subcore_barrier()

        def stage_idx(c, b):
            e0 = abase + c * K
            pltpu.async_copy(src_hbm.at[pl.ds(e0, K)], srcb[b], isem[b])
            pltpu.async_copy(dst_hbm.at[pl.ds(e0, K)], dstb[b], isem[b])

        def wait_idx(b):
            pltpu.make_async_copy(src_hbm.at[pl.ds(0, K)], srcb[b],
                                  isem[b]).wait()
            pltpu.make_async_copy(dst_hbm.at[pl.ds(0, K)], dstb[b],
                                  isem[b]).wait()

        def start_gather(b):
            pltpu.async_copy(y_hbm.at[srcb[b]], rowsb[b], gsem[b])

        def wait_gather(b):
            pltpu.make_async_copy(y_hbm.at[srcb[b]], rowsb[b],
                                  gsem[b]).wait()

        def start_scatter(b):
            pltpu.async_copy(rowsb[b], acc_sh.at[dstb[b]], ssem[b],
                             add=True)

        def wait_scatter(b):
            pltpu.make_async_copy(rowsb[b], acc_sh.at[dstb[b]],
                                  ssem[b]).wait()

        with jax.named_scope("ph_agg"):
            stage_idx(0, 0)
            wait_idx(0)
            start_gather(0)
            stage_idx(1, 1)
            wait_gather(0)
            start_scatter(0)

            @pl.loop(0, (ch - 2) // 2)
            def _(g):
                c = 1 + 2 * g
                for b, cc_ in ((1, c), (0, c + 1)):
                    wait_idx(b)
                    start_gather(b)
                    wait_scatter(1 - b)
                    stage_idx(cc_ + 1, 1 - b)
                    wait_gather(b)
                    start_scatter(b)

            wait_idx(1)
            start_gather(1)
            wait_scatter(0)
            wait_gather(1)
            start_scatter(1)
            wait_scatter(1)

        plsc.subcore_barrier()

        with jax.named_scope("ph_partrb"):
            readback(part_hbm)

    mesh = plsc.VectorSubcoreMesh(core_axis_name="c", subcore_axis_name="s")
    return pl.kernel(body, out_type=out_type, mesh=mesh,
                     scratch_types=scratch)


_RB = 1280


def _dot(a, b):
    return jnp.dot(a, b, preferred_element_type=F32,
                   precision=lax.Precision.HIGHEST)


def _tc_lin2(x, wl, wr, b, npad, feat):
    def body(x_ref, wl_ref, wr_ref, b_ref, y_ref, xr_ref):
        xb = x_ref[...]
        y_ref[...] = _dot(xb, wl_ref[...])
        xr_ref[...] = _dot(xb, wr_ref[...]) + b_ref[...]

    w_spec = pl.BlockSpec((feat, feat), lambda i: (0, 0))
    r_spec = pl.BlockSpec((_RB, feat), lambda i: (i, 0))
    return pl.pallas_call(
        body,
        grid=(npad // _RB,),
        in_specs=[r_spec, w_spec, w_spec,
                  pl.BlockSpec((1, feat), lambda i: (0, 0))],
        out_specs=[r_spec, r_spec],
        out_shape=[jax.ShapeDtypeStruct((npad, feat), F32)] * 2,
    )(x, wl, wr, b)


def _tc_mid(p0, p1, xr, d0, d1, wl, wr, b, npad, feat):
    def body(p0_ref, p1_ref, xr_ref, d0_ref, d1_ref, wl_ref, wr_ref, b_ref,
             y_ref, hr_ref):
        scale = 1.0 / jnp.maximum(d0_ref[...][:, :1] + d1_ref[...][:, :1],
                                  1.0)
        h = jnp.maximum((p0_ref[...] + p1_ref[...]) * scale + xr_ref[...],
                        0.0)
        y_ref[...] = _dot(h, wl_ref[...])
        hr_ref[...] = _dot(h, wr_ref[...]) + b_ref[...]

    w_spec = pl.BlockSpec((feat, feat), lambda i: (0, 0))
    r_spec = pl.BlockSpec((_RB, feat), lambda i: (i, 0))
    return pl.pallas_call(
        body,
        grid=(npad // _RB,),
        in_specs=[r_spec, r_spec, r_spec, r_spec, r_spec, w_spec, w_spec,
                  pl.BlockSpec((1, feat), lambda i: (0, 0))],
        out_specs=[r_spec, r_spec],
        out_shape=[jax.ShapeDtypeStruct((npad, feat), F32)] * 2,
    )(p0, p1, xr, d0, d1, wl, wr, b)


def _tc_out(p0, p1, hr, d0, d1, w3, b3, npad, feat, o):
    def body(p0_ref, p1_ref, hr_ref, d0_ref, d1_ref, w_ref, b_ref, o_ref):
        scale = 1.0 / jnp.maximum(d0_ref[...][:, :1] + d1_ref[...][:, :1],
                                  1.0)
        h = jnp.maximum((p0_ref[...] + p1_ref[...]) * scale + hr_ref[...],
                        0.0)
        o_ref[...] = _dot(h, w_ref[...]) + b_ref[...]

    r_spec = pl.BlockSpec((_RB, feat), lambda i: (i, 0))
    return pl.pallas_call(
        body,
        grid=(npad // _RB,),
        in_specs=[r_spec, r_spec, r_spec, r_spec, r_spec,
                  pl.BlockSpec((feat, o), lambda i: (0, 0)),
                  pl.BlockSpec((1, o), lambda i: (0, 0))],
        out_specs=pl.BlockSpec((_RB, o), lambda i: (i, 0)),
        out_shape=jax.ShapeDtypeStruct((npad, o), F32),
    )(p0, p1, hr, d0, d1, w3, b3)


@jax.jit
def kernel(x, edge_index, W_l1, b_l1, W_r1, W_l2, b_l2, W_r2, W3, b3):
    n, d = x.shape
    e = edge_index.shape[1]
    h = W_l1.shape[1]
    o = W3.shape[1]

    npad = _round_up(n + 1, NS * K)
    epad = _round_up(e, NC * NS * K * 2)

    x_pad = jnp.zeros((npad, d), F32).at[:n].set(x)
    pad_e = epad - e
    src = jnp.concatenate([edge_index[0], jnp.full((pad_e,), n, jnp.int32)])
    dst = jnp.concatenate([edge_index[1], jnp.full((pad_e,), n, jnp.int32)])

    sc1 = _make_sc_layer(npad, epad, h, True)
    sc2 = _make_sc_layer(npad, epad, h, False)

    y1, xr1 = _tc_lin2(x_pad, W_l1, W_r1, b_l1.reshape(1, -1), npad, d)
    iota = jnp.arange(npad, dtype=jnp.int32)
    part1, deg = sc1(y1, src, dst, iota)
    y2, hr2 = _tc_mid(part1[:npad], part1[npad:], xr1,
                      deg[:npad], deg[npad:],
                      W_l2, W_r2, b_l2.reshape(1, -1), npad, h)
    part2 = sc2(y2, src, dst, iota)
    if isinstance(part2, (list, tuple)):
        part2 = part2[0]
    out = _tc_out(part2[:npad], part2[npad:], hr2,
                  deg[:npad], deg[npad:],
                  W3, b3.reshape(1, -1), npad, h, o)
    return out[:n]


# speedup vs baseline: 8.1025x; 2.5212x over previous
import jax
import jax.numpy as jnp
from jax import lax
from jax.experimental import pallas as pl
from jax.experimental.pallas import tpu as pltpu
from jax.experimental.pallas import tpu_sc as plsc

NC = 2
NS = 16
K = 128

F32 = jnp.float32


def _round_up(a, b):
    return (a + b - 1) // b * b


def _make_sc_layer(npad, epad, feat, compute_deg):
    sp = npad // NS
    q = epad // (NC * NS)
    ch = q // K
    ncc = feat // 16
    npieces = sp // K

    out_type = [jax.ShapeDtypeStruct((NC * npad, feat), F32)]
    if compute_deg:
        out_type.append(jax.ShapeDtypeStruct((NC * npad, feat), F32))

    scratch = [
        pltpu.VMEM((K,), jnp.int32),
        pltpu.VMEM((K,), jnp.int32),
        pltpu.VMEM((K,), jnp.int32),
        pltpu.VMEM((K,), jnp.int32),
        pltpu.VMEM((K, feat), F32),
        pltpu.VMEM((K, feat), F32),
        pltpu.VMEM((K,), jnp.int32),
        pltpu.VMEM_SHARED((npad, feat), F32),
        pltpu.SemaphoreType.DMA,
        pltpu.SemaphoreType.DMA,
        pltpu.SemaphoreType.DMA,
        pltpu.SemaphoreType.DMA,
        pltpu.SemaphoreType.DMA,
        pltpu.SemaphoreType.DMA,
    ]

    def body(y_hbm, src_hbm, dst_hbm, iota_hbm, *rest):
        if compute_deg:
            part_hbm, deg_out = rest[0], rest[1]
            scr = rest[2:]
        else:
            part_hbm = rest[0]
            scr = rest[1:]
        (src0, src1, dst0, dst1, rows0, rows1, stg_idx, acc_sh,
         isem0, isem1, gsem0, gsem1, ssem0, ssem1) = scr
        srcb = (src0, src1)
        dstb = (dst0, dst1)
        rowsb = (rows0, rows1)
        isem = (isem0, isem1)
        gsem = (gsem0, gsem1)
        ssem = (ssem0, ssem1)

        cid = lax.axis_index("c")
        sid = lax.axis_index("s")
        r0 = sid * sp
        abase = cid * (epad // NC) + sid * q

        def set_stg(pr0):
            pltpu.sync_copy(iota_hbm.at[pl.ds(pr0, K)], stg_idx)

        def zero_own_stripe():
            @pl.loop(0, npieces)
            def _(p):
                set_stg(r0 + p * K)
                pltpu.sync_copy(rows1, acc_sh.at[stg_idx])

        def readback(out_arr):
            @pl.loop(0, npieces)
            def _(p):
                pr0 = r0 + p * K
                set_stg(pr0)
                pltpu.sync_copy(acc_sh.at[stg_idx], rows0)
                pltpu.sync_copy(rows0,
                                out_arr.at[pl.ds(cid * npad + pr0, K)])

        zf = jnp.zeros((16,), F32)
        one = jnp.ones((16,), F32)

        @pl.loop(0, K)
        def _(r):
            for cc in range(ncc):
                rows1[r, pl.ds(cc * 16, 16)] = zf
            if compute_deg:
                for cc in range(ncc):
                    rows0[r, pl.ds(cc * 16, 16)] = one

        with jax.named_scope("ph_zero"):
            zero_own_stripe()
        plsc.subcore_barrier()

        if compute_deg:
            with jax.named_scope("ph_deg"):
                @pl.loop(0, ch)
                def _(j):
                    e0 = abase + j * K
                    pltpu.sync_copy(dst_hbm.at[pl.ds(e0, K)], dst0)
                    pltpu.sync_copy(rows0, acc_sh.at[dst0], add=True)

            plsc.subcore_barrier()
            with jax.named_scope("ph_degrb"):
                readback(deg_out)
            plsc.subcore_barrier()
            with jax.named_scope("ph_rezero"):
                zero_own_stripe()
            plsc.subcore_barrier()

        def stage_idx(c, b):
            e0 = abase + c * K
            pltpu.async_copy(src_hbm.at[pl.ds(e0, K)], srcb[b], isem[b])
            pltpu.async_copy(dst_hbm.at[pl.ds(e0, K)], dstb[b], isem[b])

        def wait_idx(b):
            pltpu.make_async_copy(src_hbm.at[pl.ds(0, K)], srcb[b],
                                  isem[b]).wait()
            pltpu.make_async_copy(dst_hbm.at[pl.ds(0, K)], dstb[b],
                                  isem[b]).wait()

        def start_gather(b):
            pltpu.async_copy(y_hbm.at[srcb[b]], rowsb[b], gsem[b])

        def wait_gather(b):
            pltpu.make_async_copy(y_hbm.at[srcb[b]], rowsb[b],
                                  gsem[b]).wait()

        def start_scatter(b):
            pltpu.async_copy(rowsb[b], acc_sh.at[dstb[b]], ssem[b],
                             add=True)

        def wait_scatter(b):
            pltpu.make_async_copy(rowsb[b], acc_sh.at[dstb[b]],
                                  ssem[b]).wait()

        with jax.named_scope("ph_agg"):
            stage_idx(0, 0)
            wait_idx(0)
            start_gather(0)
            stage_idx(1, 1)
            wait_gather(0)
            start_scatter(0)

            @pl.loop(0, (ch - 2) // 2)
            def _(g):
                c = 1 + 2 * g
                for b, cc_ in ((1, c), (0, c + 1)):
                    wait_idx(b)
                    start_gather(b)
                    wait_scatter(1 - b)
                    stage_idx(cc_ + 1, 1 - b)
                    wait_gather(b)
                    start_scatter(b)

            wait_idx(1)
            start_gather(1)
            wait_scatter(0)
            wait_gather(1)
            start_scatter(1)
            wait_scatter(1)

        plsc.subcore_barrier()

        with jax.named_scope("ph_partrb"):
            readback(part_hbm)

    mesh = plsc.VectorSubcoreMesh(core_axis_name="c", subcore_axis_name="s")
    return pl.kernel(body, out_type=out_type, mesh=mesh,
                     scratch_types=scratch)


_RB = 1280


def _dot(a, b):
    return jnp.dot(a, b, preferred_element_type=F32,
                   precision=lax.Precision.HIGHEST)


def _tc_lin2(x, wl, wr, b, npad, feat):
    def body(x_ref, wl_ref, wr_ref, b_ref, y_ref, xr_ref):
        xb = x_ref[...]
        y_ref[...] = _dot(xb, wl_ref[...])
        xr_ref[...] = _dot(xb, wr_ref[...]) + b_ref[...]

    w_spec = pl.BlockSpec((feat, feat), lambda i: (0, 0))
    r_spec = pl.BlockSpec((_RB, feat), lambda i: (i, 0))
    return pl.pallas_call(
        body,
        grid=(npad // _RB,),
        in_specs=[r_spec, w_spec, w_spec,
                  pl.BlockSpec((1, feat), lambda i: (0, 0))],
        out_specs=[r_spec, r_spec],
        out_shape=[jax.ShapeDtypeStruct((npad, feat), F32)] * 2,
    )(x, wl, wr, b)


def _tc_mid(p0, p1, xr, d0, d1, wl, wr, b, npad, feat):
    def body(p0_ref, p1_ref, xr_ref, d0_ref, d1_ref, wl_ref, wr_ref, b_ref,
             y_ref, hr_ref):
        scale = 1.0 / jnp.maximum(d0_ref[...][:, :1] + d1_ref[...][:, :1],
                                  1.0)
        h = jnp.maximum((p0_ref[...] + p1_ref[...]) * scale + xr_ref[...],
                        0.0)
        y_ref[...] = _dot(h, wl_ref[...])
        hr_ref[...] = _dot(h, wr_ref[...]) + b_ref[...]

    w_spec = pl.BlockSpec((feat, feat), lambda i: (0, 0))
    r_spec = pl.BlockSpec((_RB, feat), lambda i: (i, 0))
    return pl.pallas_call(
        body,
        grid=(npad // _RB,),
        in_specs=[r_spec, r_spec, r_spec, r_spec, r_spec, w_spec, w_spec,
                  pl.BlockSpec((1, feat), lambda i: (0, 0))],
        out_specs=[r_spec, r_spec],
        out_shape=[jax.ShapeDtypeStruct((npad, feat), F32)] * 2,
    )(p0, p1, xr, d0, d1, wl, wr, b)


def _tc_out(p0, p1, hr, d0, d1, w3, b3, npad, feat, o):
    def body(p0_ref, p1_ref, hr_ref, d0_ref, d1_ref, w_ref, b_ref, o_ref):
        scale = 1.0 / jnp.maximum(d0_ref[...][:, :1] + d1_ref[...][:, :1],
                                  1.0)
        h = jnp.maximum((p0_ref[...] + p1_ref[...]) * scale + hr_ref[...],
                        0.0)
        o_ref[...] = _dot(h, w_ref[...]) + b_ref[...]

    r_spec = pl.BlockSpec((_RB, feat), lambda i: (i, 0))
    return pl.pallas_call(
        body,
        grid=(npad // _RB,),
        in_specs=[r_spec, r_spec, r_spec, r_spec, r_spec,
                  pl.BlockSpec((feat, o), lambda i: (0, 0)),
                  pl.BlockSpec((1, o), lambda i: (0, 0))],
        out_specs=pl.BlockSpec((_RB, o), lambda i: (i, 0)),
        out_shape=jax.ShapeDtypeStruct((npad, o), F32),
    )(p0, p1, hr, d0, d1, w3, b3)


@jax.jit
def kernel(x, edge_index, W_l1, b_l1, W_r1, W_l2, b_l2, W_r2, W3, b3):
    n, d = x.shape
    e = edge_index.shape[1]
    h = W_l1.shape[1]
    o = W3.shape[1]

    npad = _round_up(n + 1, NS * K)
    epad = _round_up(e, NC * NS * K * 2)

    x_pad = jnp.zeros((npad, d), F32).at[:n].set(x)
    pad_e = epad - e
    junk = n + jnp.arange(pad_e, dtype=jnp.int32) % (npad - n)
    src = jnp.concatenate([edge_index[0], junk])
    dst = jnp.concatenate([edge_index[1], junk])

    sc1 = _make_sc_layer(npad, epad, h, True)
    sc2 = _make_sc_layer(npad, epad, h, False)

    y1, xr1 = _tc_lin2(x_pad, W_l1, W_r1, b_l1.reshape(1, -1), npad, d)
    iota = jnp.arange(npad, dtype=jnp.int32)
    part1, deg = sc1(y1, src, dst, iota)
    y2, hr2 = _tc_mid(part1[:npad], part1[npad:], xr1,
                      deg[:npad], deg[npad:],
                      W_l2, W_r2, b_l2.reshape(1, -1), npad, h)
    part2 = sc2(y2, src, dst, iota)
    if isinstance(part2, (list, tuple)):
        part2 = part2[0]
    out = _tc_out(part2[:npad], part2[npad:], hr2,
                  deg[:npad], deg[npad:],
                  W3, b3.reshape(1, -1), npad, h, o)
    return out[:n]
